# Initial kernel scaffold; baseline (speedup 1.0000x reference)
import jax
import jax.numpy as jnp
from jax.experimental import pallas as pl


def kernel(content_input, W, b):
    raise NotImplementedError("write your pallas kernel here")


# speedup vs baseline: 4.7444x; 4.7444x over previous
import functools

import jax
import jax.numpy as jnp
from jax import lax
from jax.experimental import pallas as pl
from jax.experimental.pallas import tpu as pltpu
from jax.experimental.pallas import tpu_sc as plsc

B, L, H = 4096, 20, 8
VOCAB, EMBED = 1000, 64
N = B * L
NC, NS = 2, 16
NW = NC * NS
BAGS_PER_W = N // NW
CHUNK = 512
NCHUNK = BAGS_PER_W // CHUNK
TABLE_WORDS = EMBED * VOCAB


def _sc_kernel(table_hbm, idx_hbm, out_hbm, table_v, idx_v, out_v):
  wid = lax.axis_index("s") * NC + lax.axis_index("c")
  pltpu.sync_copy(table_hbm, table_v)

  iota = jnp.arange(16, dtype=jnp.int32)
  lane8 = iota * 8
  lane64 = iota * 64

  def chunk_body(ci, carry):
    bag0 = wid * BAGS_PER_W + ci * CHUNK
    pltpu.sync_copy(idx_hbm.at[pl.ds(bag0 * 8, CHUNK * 8)], idx_v)

    def group_body(g, carry2):
      gi = g * 128
      idx_h = [plsc.load_gather(idx_v, [lane8 + (gi + h)]) for h in range(H)]
      go = g * (16 * EMBED)
      for e in range(EMBED):
        acc = plsc.load_gather(table_v, [idx_h[0] + (e * VOCAB)])
        for h in range(1, H):
          acc = acc + plsc.load_gather(table_v, [idx_h[h] + (e * VOCAB)])
        plsc.store_scatter(out_v, [lane64 + (go + e)], acc)
      return carry2

    lax.fori_loop(0, CHUNK // 16, group_body, 0, unroll=False)
    pltpu.sync_copy(out_v, out_hbm.at[pl.ds(bag0 * EMBED, CHUNK * EMBED)])
    return carry

  lax.fori_loop(0, NCHUNK, chunk_body, 0, unroll=False)


@jax.jit
def kernel(content_input, W, b):
  idx = content_input.astype(jnp.int32).reshape(-1)
  table = (W + b[:, None] * (1.0 / H)).reshape(-1)
  run = pl.kernel(
      _sc_kernel,
      out_type=jax.ShapeDtypeStruct((N * EMBED,), jnp.float32),
      mesh=plsc.VectorSubcoreMesh(
          core_axis_name="c", subcore_axis_name="s", num_cores=NC,
          num_subcores=NS),
      scratch_types=[
          pltpu.VMEM((TABLE_WORDS,), jnp.float32),
          pltpu.VMEM((CHUNK * H,), jnp.int32),
          pltpu.VMEM((CHUNK * EMBED,), jnp.float32),
      ],
      compiler_params=pltpu.CompilerParams(needs_layout_passes=False),
  )
  out = run(table, idx)
  return out.reshape(B, L, EMBED)
